# baseline (speedup 1.0000x reference)
import functools

import jax
import jax.numpy as jnp
from jax import lax
from jax.experimental import pallas as pl
from jax.experimental.pallas import tpu as pltpu
from jax.experimental.pallas import tpu_sc as plsc

D_MODEL = 128
HALF = D_MODEL // 2
N = 327680

NC = 2
NS = 16
NW = NC * NS
ROWS_PER_W = N // NW
CHUNK = 512
SUB = CHUNK // 128
N_CHUNKS = ROWS_PER_W // CHUNK
STRIP = CHUNK + 128
SSUB = STRIP // 128


def _sc_body(torder_hbm, ttable_hbm, qtable_hbm, nq_hbm, out_hbm,
             tidx_v, sidx_v, tbuf_v, qstrip_v, nq_v, ttab_sh, gsem, isem,
             wsem, ssem):
    wid = lax.axis_index("s") * NC + lax.axis_index("c")
    wbase = wid * ROWS_PER_W

    @pl.when(lax.axis_index("s") == 0)
    def _stage_table():
        pltpu.sync_copy(ttable_hbm, ttab_sh)
    plsc.subcore_barrier()

    pltpu.sync_copy(nq_hbm, nq_v)
    nq_i = nq_v[...]
    nq_f = nq_i.astype(jnp.float32)
    nq = nq_i[0]
    lane = jax.lax.iota(jnp.int32, 16)

    for v in range(STRIP // 16):
        jvec = v * 16 + lane
        t = (jvec.astype(jnp.float32) / nq_f).astype(jnp.int32)
        r = jvec - t * nq_i
        r = jnp.where(r < 0, r + nq_i, r)
        r = jnp.where(r >= nq_i, r - nq_i, r)
        sidx_v[v // 8, pl.ds((v % 8) * 16, 16)] = r
    strip_copies = []
    for k in range(SSUB):
        strip_copies.append(pltpu.async_copy(
            qtable_hbm.at[sidx_v.at[k]],
            qstrip_v.at[pl.ds(k * 128, 128)], ssem))
    for c in strip_copies:
        c.wait()

    def stage_idx(ci, s):
        base = wbase + ci * CHUNK
        for j in range(SUB):
            pltpu.async_copy(torder_hbm.at[pl.ds(base + j * 128, 128)],
                             tidx_v.at[s, j], isem.at[s])

    def wait_idx(ci, s):
        base = wbase + ci * CHUNK
        for j in range(SUB):
            pltpu.make_async_copy(torder_hbm.at[pl.ds(base + j * 128, 128)],
                                  tidx_v.at[s, j], isem.at[s]).wait()

    def fire_gathers(s):
        for j in range(SUB):
            pltpu.async_copy(
                ttab_sh.at[tidx_v.at[s, j]],
                tbuf_v.at[s, pl.ds(j * 128, 128)], gsem.at[s])

    def wait_gathers(s):
        for j in range(SUB):
            pltpu.make_async_copy(
                ttab_sh.at[tidx_v.at[s, j]],
                tbuf_v.at[s, pl.ds(j * 128, 128)], gsem.at[s]).wait()

    HC = CHUNK // 2

    def fire_write(ci, s):
        base = wbase + ci * CHUNK
        off = lax.rem(base, nq)
        for h in range(2):
            pltpu.async_copy(
                tbuf_v.at[s, pl.ds(h * HC, HC)],
                out_hbm.at[pl.ds(base + h * HC, HC), pl.ds(0, HALF)],
                wsem.at[s])
            pltpu.async_copy(
                qstrip_v.at[pl.ds(off + h * HC, HC)],
                out_hbm.at[pl.ds(base + h * HC, HC), pl.ds(HALF, HALF)],
                wsem.at[s])

    def wait_write(ci, s):
        base = wbase + ci * CHUNK
        off = lax.rem(base, nq)
        for h in range(2):
            pltpu.make_async_copy(
                tbuf_v.at[s, pl.ds(h * HC, HC)],
                out_hbm.at[pl.ds(base + h * HC, HC), pl.ds(0, HALF)],
                wsem.at[s]).wait()
            pltpu.make_async_copy(
                qstrip_v.at[pl.ds(off + h * HC, HC)],
                out_hbm.at[pl.ds(base + h * HC, HC), pl.ds(HALF, HALF)],
                wsem.at[s]).wait()

    stage_idx(0, 0)
    stage_idx(1, 1)
    wait_idx(0, 0)
    fire_gathers(0)

    def loop_body(ci, carry):
        s = lax.bitwise_and(ci, 1)
        s1 = 1 - s

        wait_gathers(s)

        @pl.when(ci + 2 < N_CHUNKS)
        def _stage_next_idx():
            stage_idx(ci + 2, s)

        @pl.when(ci + 1 < N_CHUNKS)
        def _launch_next():
            wait_idx(ci + 1, s1)
            @pl.when(ci >= 1)
            def _drain_prev_write():
                wait_write(ci - 1, s1)
            fire_gathers(s1)

        fire_write(ci, s)
        return carry

    lax.fori_loop(0, N_CHUNKS, loop_body, 0)
    wait_write(N_CHUNKS - 2, (N_CHUNKS - 2) % 2)
    wait_write(N_CHUNKS - 1, (N_CHUNKS - 1) % 2)


@jax.jit
def _call(temporal_order, temporal_table, qubit_table, nq16):
    mesh = plsc.VectorSubcoreMesh(core_axis_name="c", subcore_axis_name="s")
    f = pl.kernel(
        _sc_body,
        mesh=mesh,
        compiler_params=pltpu.CompilerParams(use_tc_tiling_on_sc=False),
        out_type=jax.ShapeDtypeStruct((N, D_MODEL), jnp.float32),
        scratch_types=[
            pltpu.VMEM((2, SUB, 128), jnp.int32),
            pltpu.VMEM((SSUB, 128), jnp.int32),
            pltpu.VMEM((2, CHUNK, HALF), jnp.float32),
            pltpu.VMEM((STRIP, HALF), jnp.float32),
            pltpu.VMEM((16,), jnp.int32),
            pltpu.VMEM_SHARED((1000, HALF), jnp.float32),
            pltpu.SemaphoreType.DMA((2,)),
            pltpu.SemaphoreType.DMA((2,)),
            pltpu.SemaphoreType.DMA((2,)),
            pltpu.SemaphoreType.DMA,
        ],
    )
    return f(temporal_order, temporal_table, qubit_table, nq16)


def kernel(temporal_order, grid_shape, temporal_table, qubit_table):
    nq16 = jnp.broadcast_to(grid_shape[1].astype(jnp.int32), (16,))
    return _call(temporal_order.astype(jnp.int32), temporal_table,
                 qubit_table, nq16)


# speedup vs baseline: 1.0041x; 1.0041x over previous
import functools

import jax
import jax.numpy as jnp
from jax import lax
from jax.experimental import pallas as pl
from jax.experimental.pallas import tpu as pltpu
from jax.experimental.pallas import tpu_sc as plsc

D_MODEL = 128
HALF = D_MODEL // 2
N = 327680

NC = 2
NS = 16
NW = NC * NS
ROWS_PER_W = N // NW
CHUNK = 512
SUB = CHUNK // 128
N_CHUNKS = ROWS_PER_W // CHUNK
STRIP = CHUNK + 128
SSUB = STRIP // 128


def _sc_body(torder_hbm, ttable_hbm, qtable_hbm, nq_hbm, out_hbm,
             tidx_v, sidx_v, tbuf_v, qstrip_v, nq_v, ttab_sh, gsem, isem,
             wsem, ssem):
    wid = lax.axis_index("s") * NC + lax.axis_index("c")
    wbase = wid * ROWS_PER_W

    @pl.when(lax.axis_index("s") == 0)
    def _stage_table():
        pltpu.sync_copy(ttable_hbm, ttab_sh)
    plsc.subcore_barrier()

    pltpu.sync_copy(nq_hbm, nq_v)
    nq_i = nq_v[...]
    nq_f = nq_i.astype(jnp.float32)
    nq = nq_i[0]
    lane = jax.lax.iota(jnp.int32, 16)

    for v in range(STRIP // 16):
        jvec = v * 16 + lane
        t = (jvec.astype(jnp.float32) / nq_f).astype(jnp.int32)
        r = jvec - t * nq_i
        r = jnp.where(r < 0, r + nq_i, r)
        r = jnp.where(r >= nq_i, r - nq_i, r)
        sidx_v[v // 8, pl.ds((v % 8) * 16, 16)] = r
    strip_copies = []
    for k in range(SSUB):
        strip_copies.append(pltpu.async_copy(
            qtable_hbm.at[sidx_v.at[k]],
            qstrip_v.at[pl.ds(k * 128, 128)], ssem))
    for c in strip_copies:
        c.wait()

    def stage_idx(ci, s):
        base = wbase + ci * CHUNK
        for j in range(SUB):
            pltpu.async_copy(torder_hbm.at[pl.ds(base + j * 128, 128)],
                             tidx_v.at[s, j], isem.at[s])

    def wait_idx(ci, s):
        base = wbase + ci * CHUNK
        for j in range(SUB):
            pltpu.make_async_copy(torder_hbm.at[pl.ds(base + j * 128, 128)],
                                  tidx_v.at[s, j], isem.at[s]).wait()

    def fire_gathers(s):
        for j in range(SUB):
            pltpu.async_copy(
                ttab_sh.at[tidx_v.at[s, j]],
                tbuf_v.at[s, pl.ds(j * 128, 128)], gsem.at[s])

    def wait_gathers(s):
        for j in range(SUB):
            pltpu.make_async_copy(
                ttab_sh.at[tidx_v.at[s, j]],
                tbuf_v.at[s, pl.ds(j * 128, 128)], gsem.at[s]).wait()

    def fire_write(ci, s):
        base = wbase + ci * CHUNK
        off = lax.rem(base, nq)
        pltpu.async_copy(
            tbuf_v.at[s],
            out_hbm.at[pl.ds(base, CHUNK), pl.ds(0, HALF)], wsem.at[s])
        pltpu.async_copy(
            qstrip_v.at[pl.ds(off, CHUNK)],
            out_hbm.at[pl.ds(base, CHUNK), pl.ds(HALF, HALF)], wsem.at[s])

    def wait_write(ci, s):
        base = wbase + ci * CHUNK
        off = lax.rem(base, nq)
        pltpu.make_async_copy(
            tbuf_v.at[s],
            out_hbm.at[pl.ds(base, CHUNK), pl.ds(0, HALF)], wsem.at[s]).wait()
        pltpu.make_async_copy(
            qstrip_v.at[pl.ds(off, CHUNK)],
            out_hbm.at[pl.ds(base, CHUNK), pl.ds(HALF, HALF)],
            wsem.at[s]).wait()

    stage_idx(0, 0)
    stage_idx(1, 1)
    wait_idx(0, 0)
    fire_gathers(0)

    def loop_body(ci, carry):
        s = lax.bitwise_and(ci, 1)
        s1 = 1 - s

        wait_gathers(s)

        @pl.when(ci + 2 < N_CHUNKS)
        def _stage_next_idx():
            stage_idx(ci + 2, s)

        @pl.when(ci + 1 < N_CHUNKS)
        def _launch_next():
            wait_idx(ci + 1, s1)
            @pl.when(ci >= 1)
            def _drain_prev_write():
                wait_write(ci - 1, s1)
            fire_gathers(s1)

        fire_write(ci, s)
        return carry

    lax.fori_loop(0, N_CHUNKS, loop_body, 0)
    wait_write(N_CHUNKS - 2, (N_CHUNKS - 2) % 2)
    wait_write(N_CHUNKS - 1, (N_CHUNKS - 1) % 2)


@jax.jit
def _call(temporal_order, temporal_table, qubit_table, nq16):
    mesh = plsc.VectorSubcoreMesh(core_axis_name="c", subcore_axis_name="s")
    f = pl.kernel(
        _sc_body,
        mesh=mesh,
        compiler_params=pltpu.CompilerParams(use_tc_tiling_on_sc=False),
        out_type=jax.ShapeDtypeStruct((N, D_MODEL), jnp.float32),
        scratch_types=[
            pltpu.VMEM((2, SUB, 128), jnp.int32),
            pltpu.VMEM((SSUB, 128), jnp.int32),
            pltpu.VMEM((2, CHUNK, HALF), jnp.float32),
            pltpu.VMEM((STRIP, HALF), jnp.float32),
            pltpu.VMEM((16,), jnp.int32),
            pltpu.VMEM_SHARED((1000, HALF), jnp.float32),
            pltpu.SemaphoreType.DMA((2,)),
            pltpu.SemaphoreType.DMA((2,)),
            pltpu.SemaphoreType.DMA((2,)),
            pltpu.SemaphoreType.DMA,
        ],
    )
    return f(temporal_order, temporal_table, qubit_table, nq16)


def kernel(temporal_order, grid_shape, temporal_table, qubit_table):
    nq16 = jnp.broadcast_to(grid_shape[1].astype(jnp.int32), (16,))
    return _call(temporal_order.astype(jnp.int32), temporal_table,
                 qubit_table, nq16)
